# Initial kernel scaffold; baseline (speedup 1.0000x reference)
import jax
import jax.numpy as jnp
from jax.experimental import pallas as pl


def kernel(x, edge_index, W, b, gamma, beta):
    raise NotImplementedError("write your pallas kernel here")


# speedup vs baseline: 1.6453x; 1.6453x over previous
import jax
import jax.numpy as jnp
from jax import lax
from jax.experimental import pallas as pl
from jax.experimental.pallas import tpu as pltpu
from jax.experimental.pallas import tpu_sc as plsc

N = 10000
E = 160000
D = 256

NC = 2
NS = 16
NW = NC * NS
OWN = 320
DUMMY = OWN
ACC_ROWS = OWN + 16
CE = 2000
NQ = E // CE
NGR = CE // 16
GB = 128
PEND = 256
NPAD = N + NW * OWN - N
OUT_ROWS = NW * OWN


def _sc_body(x_ref, src_ref, dst_ref, agg_out, deg_out,
             src_v, dst_v, pend, gidx, rows_v, acc, dega, sem):
    c = lax.axis_index("c")
    s = lax.axis_index("s")
    w = s * NC + c
    lo = w * OWN

    zf = jnp.zeros((16,), jnp.float32)
    zi = jnp.zeros((16,), jnp.int32)
    oneh = (lax.iota(jnp.int32, 16) < 1).astype(jnp.int32)
    dummy_v = jnp.full((16,), DUMMY, jnp.int32)

    def zacc(r, carry):
        for k in range(D // 16):
            acc[r, pl.ds(k * 16, 16)] = zf
        return carry
    lax.fori_loop(0, ACC_ROWS, zacc, 0)
    for k in range(ACC_ROWS // 16 + 1):
        dega[pl.ds(k * 16, 16)] = zi

    def flush(base):
        for kk in range(GB // 16):
            v = pend[pl.ds(base + kk * 16, 16)]
            gidx[pl.ds(kk * 16, 16)] = v >> 9
        pltpu.async_copy(x_ref.at[gidx], rows_v, sem).wait()

        def accrow(r, carry):
            ldst = pend[pl.ds(base + r, 16)][0] & 511
            for k in range(D // 16):
                plsc.addupdate(acc.at[ldst, pl.ds(k * 16, 16)],
                               rows_v[r, pl.ds(k * 16, 16)])
            plsc.addupdate(dega.at[pl.ds(ldst, 16)], oneh)
            return carry
        lax.fori_loop(0, GB, accrow, 0)

    def chunk(q, cnt):
        eb = q * CE
        pltpu.sync_copy(src_ref.at[pl.ds(eb, CE)], src_v)
        pltpu.sync_copy(dst_ref.at[pl.ds(eb, CE)], dst_v)

        def group(i, cnt):
            vd = dst_v[pl.ds(i * 16, 16)]
            vs = src_v[pl.ds(i * 16, 16)]
            d2 = vd - lo
            clp = jnp.minimum(jnp.maximum(d2, 0), OWN - 1)
            mi = 1 - jnp.minimum(jnp.abs(d2 - clp), 1)
            h = plsc.cumsum(mi)[15]
            packed = vs * 512 + jnp.where(mi == 1, clp, DUMMY)
            _, srt = plsc.sort_key_val(1 - mi, packed)
            pend[pl.ds(cnt, 16)] = srt
            cnt = cnt + h

            @pl.when(cnt >= GB)
            def _():
                flush(0)
                pend[pl.ds(0, 16)] = pend[pl.ds(GB, 16)]
            cnt = jnp.where(cnt >= GB, cnt - GB, cnt)
            return cnt
        return lax.fori_loop(0, NGR, group, cnt)

    cnt = lax.fori_loop(0, NQ, chunk, jnp.int32(0))

    for k in range(GB // 16):
        pend[pl.ds(cnt + k * 16, 16)] = dummy_v

    @pl.when(cnt > 0)
    def _():
        flush(0)

    pltpu.sync_copy(acc.at[pl.ds(0, OWN)], agg_out.at[pl.ds(lo, OWN)])
    pltpu.sync_copy(dega.at[pl.ds(0, OWN)], deg_out.at[pl.ds(lo, OWN)])


_sc_aggregate = pl.kernel(
    _sc_body,
    out_type=(
        jax.ShapeDtypeStruct((OUT_ROWS, D), jnp.float32),
        jax.ShapeDtypeStruct((OUT_ROWS,), jnp.int32),
    ),
    mesh=plsc.VectorSubcoreMesh(core_axis_name="c", subcore_axis_name="s"),
    compiler_params=pltpu.CompilerParams(needs_layout_passes=False),
    scratch_types=(
        pltpu.VMEM((CE,), jnp.int32),
        pltpu.VMEM((CE,), jnp.int32),
        pltpu.VMEM((PEND,), jnp.int32),
        pltpu.VMEM((GB,), jnp.int32),
        pltpu.VMEM((GB, D), jnp.float32),
        pltpu.VMEM((ACC_ROWS, D), jnp.float32),
        pltpu.VMEM((ACC_ROWS + 16,), jnp.int32),
        pltpu.SemaphoreType.DMA,
    ),
)


BN = 400


def _tc_body(deg_ref, agg_ref, w_ref, b_ref, g_ref, be_ref, o_ref):
    d = deg_ref[...].astype(jnp.float32)
    a = agg_ref[...]
    h = a / jnp.maximum(d, 1.0)
    h = jnp.dot(h, w_ref[...], preferred_element_type=jnp.float32)
    h = h + b_ref[...]
    mu = jnp.mean(h, axis=1, keepdims=True)
    var = jnp.mean((h - mu) ** 2, axis=1, keepdims=True)
    h = (h - mu) * lax.rsqrt(var + 1e-5)
    h = h * g_ref[...] + be_ref[...]
    o_ref[...] = jnp.maximum(h, 0.0)


def _tc_dense(degp, aggp, W, b, gamma, beta):
    return pl.pallas_call(
        _tc_body,
        grid=(N // BN,),
        in_specs=[
            pl.BlockSpec((BN, 1), lambda i: (i, 0)),
            pl.BlockSpec((BN, D), lambda i: (i, 0)),
            pl.BlockSpec((D, D), lambda i: (0, 0)),
            pl.BlockSpec((1, D), lambda i: (0, 0)),
            pl.BlockSpec((1, D), lambda i: (0, 0)),
            pl.BlockSpec((1, D), lambda i: (0, 0)),
        ],
        out_specs=pl.BlockSpec((BN, D), lambda i: (i, 0)),
        out_shape=jax.ShapeDtypeStruct((N, D), jnp.float32),
    )(degp, aggp, W, b, gamma, beta)


def kernel(x, edge_index, W, b, gamma, beta):
    src = edge_index[0]
    dst = edge_index[1]
    aggp, degp = _sc_aggregate(x, src, dst)
    return _tc_dense(degp[:, None], aggp, W,
                     b[None, :], gamma[None, :], beta[None, :])
